# baseline (speedup 1.0000x reference)
import functools

import jax
import jax.numpy as jnp
from jax import lax
from jax.experimental import pallas as pl
from jax.experimental.pallas import tpu as pltpu
from jax.experimental.pallas import tpu_sc as plsc

_NUM_FIELDS = 26
_FIELD_SIZE = 100000
_EMBED_DIM = 32
_BATCH = 16384
_TOTAL = _BATCH * _NUM_FIELDS
_CHUNK = 128
_NCHUNKS = _TOTAL // _CHUNK
_NC = 2
_NS = 16
_NW = _NC * _NS
_CPW = _NCHUNKS // _NW
_NBUF = 8
_GROUPS = _CPW // _NBUF
_LANES = 16


def _make_sc_gather():
    mesh = plsc.VectorSubcoreMesh(core_axis_name="c", subcore_axis_name="s")

    @functools.partial(
        pl.kernel,
        mesh=mesh,
        out_type=jax.ShapeDtypeStruct((_NCHUNKS, _CHUNK, _EMBED_DIM), jnp.float32),
        compiler_params=pltpu.CompilerParams(use_tc_tiling_on_sc=False),
        scratch_types=(
            [
                pltpu.VMEM((_CPW, _CHUNK), jnp.int32),
                pltpu.VMEM((_NBUF, _CHUNK, _EMBED_DIM), jnp.float32),
            ]
            + [pltpu.SemaphoreType.DMA] * _NBUF
        ),
    )
    def k(x_hbm, tbl_hbm, out_hbm, idx_v, rows_v, *gsems):
        wid = lax.axis_index("s") * _NC + lax.axis_index("c")
        pltpu.sync_copy(x_hbm.at[wid], idx_v)

        def add_offsets(c):
            for kk in range(_CHUNK // _LANES):
                pos = c * _CHUNK + kk * _LANES + lax.iota(jnp.int32, _LANES)
                col = lax.rem(pos, _NUM_FIELDS)
                sl = pl.ds(kk * _LANES, _LANES)
                idx_v[c, sl] = idx_v[c, sl] + col * _FIELD_SIZE

        def fire_gather(c, b):
            pltpu.async_copy(tbl_hbm.at[idx_v.at[c]], rows_v.at[b], gsems[b])

        def wait_gather(c, b):
            pltpu.make_async_copy(
                tbl_hbm.at[idx_v.at[c]], rows_v.at[b], gsems[b]
            ).wait()

        def drain(c, b):
            wait_gather(c, b)
            pltpu.sync_copy(rows_v.at[b], out_hbm.at[wid * _CPW + c])

        for b in range(_NBUF):
            add_offsets(b)
            fire_gather(b, b)

        def group(g, carry):
            for b in range(_NBUF):
                c = g * _NBUF + b
                drain(c, b)
                add_offsets(c + _NBUF)
                fire_gather(c + _NBUF, b)
            return carry

        lax.fori_loop(0, _GROUPS - 1, group, 0)

        for b in range(_NBUF):
            drain((_GROUPS - 1) * _NBUF + b, b)

    return k


_sc_gather = _make_sc_gather()


def kernel(x, table):
    x_blocks = x.astype(jnp.int32).reshape(_NW, _CPW, _CHUNK)
    out = _sc_gather(x_blocks, table)
    return out.reshape(_BATCH, _NUM_FIELDS, _EMBED_DIM)


# speedup vs baseline: 1.0003x; 1.0003x over previous
import functools

import jax
import jax.numpy as jnp
from jax import lax
from jax.experimental import pallas as pl
from jax.experimental.pallas import tpu as pltpu
from jax.experimental.pallas import tpu_sc as plsc

_NUM_FIELDS = 26
_FIELD_SIZE = 100000
_EMBED_DIM = 32
_BATCH = 16384
_TOTAL = _BATCH * _NUM_FIELDS
_CHUNK = 128
_NCHUNKS = _TOTAL // _CHUNK
_NC = 2
_NS = 16
_NW = _NC * _NS
_CPW = _NCHUNKS // _NW
_NBUF = 4
_GROUPS = _CPW // _NBUF
_LANES = 16
_PADDED = 128


def _make_sc_gather():
    mesh = plsc.VectorSubcoreMesh(core_axis_name="c", subcore_axis_name="s")

    @functools.partial(
        pl.kernel,
        mesh=mesh,
        out_type=jax.ShapeDtypeStruct(
            (_NUM_FIELDS, 4, 128, 8, _CHUNK), jnp.float32
        ),
        compiler_params=pltpu.CompilerParams(
            use_tc_tiling_on_sc=False, needs_layout_passes=False
        ),
        scratch_types=(
            [
                pltpu.VMEM((_CPW, _CHUNK), jnp.int32),
            ]
            + [pltpu.VMEM((_CHUNK, _PADDED), jnp.float32)] * _NBUF
            + [pltpu.VMEM((2, 32, _CHUNK), jnp.float32)]
            + [pltpu.SemaphoreType.DMA] * (_NBUF + 2)
        ),
    )
    def k(x_hbm, tbl_hbm, out_hbm, idx_v, *bufs_and_sems):
        rows_v = bufs_and_sems[:_NBUF]
        obuf_v = bufs_and_sems[_NBUF]
        sems = bufs_and_sems[_NBUF + 1:]
        gsems = sems[:_NBUF]
        osems = sems[_NBUF:]
        wid = lax.axis_index("s") * _NC + lax.axis_index("c")
        g0 = wid * _CPW
        pltpu.sync_copy(x_hbm.at[wid], idx_v)

        rowidx = [lax.iota(jnp.int32, _LANES) + _LANES * kk for kk in range(8)]
        zeros = jnp.zeros((_LANES,), jnp.int32)

        def add_offsets(c, f):
            off = zeros + f * _FIELD_SIZE
            for kk in range(_CHUNK // _LANES):
                sl = pl.ds(kk * _LANES, _LANES)
                idx_v[c, sl] = idx_v[c, sl] + off

        def fire_gather(c, b):
            pltpu.async_copy(tbl_hbm.at[idx_v.at[c]], rows_v[b], gsems[b])

        def wait_gather(c, b):
            pltpu.make_async_copy(
                tbl_hbm.at[idx_v.at[c]], rows_v[b], gsems[b]
            ).wait()

        def drain(c, b, p, f, bt):
            wait_gather(c, b)

            def tj(j, carry):
                col = zeros + j
                for kk in range(8):
                    v = plsc.load_gather(rows_v[b], [rowidx[kk], col])
                    obuf_v[p, j, pl.ds(kk * _LANES, _LANES)] = v
                return carry

            lax.fori_loop(0, 32, tj, 0)
            for cg in range(4):
                pltpu.async_copy(
                    obuf_v.at[p, pl.ds(8 * cg, 8)],
                    out_hbm.at[f, cg, bt],
                    osems[p],
                )

        def wait_out(p, f, bt):
            for cg in range(4):
                pltpu.make_async_copy(
                    obuf_v.at[p, pl.ds(8 * cg, 8)],
                    out_hbm.at[f, cg, bt],
                    osems[p],
                ).wait()

        def fb(g):
            return g // _CHUNK, lax.rem(g, _CHUNK)

        for b in range(_NBUF):
            add_offsets(b, (g0 + b) // _CHUNK)
            fire_gather(b, b)

        for c in range(_NBUF):
            g = g0 + c
            if c >= 2:
                fq, btq = fb(g - 2)
                wait_out(c % 2, fq, btq)
            add_offsets(c + _NBUF, (g + _NBUF) // _CHUNK)
            f, bt = fb(g)
            drain(c, c % _NBUF, c % 2, f, bt)
            fire_gather(c + _NBUF, c % _NBUF)

        def steady(gr, carry):
            for b in range(_NBUF):
                c = gr * _NBUF + b
                g = g0 + c
                fq, btq = fb(g - 2)
                wait_out(b % 2, fq, btq)
                add_offsets(c + _NBUF, (g + _NBUF) // _CHUNK)
                f, bt = fb(g)
                drain(c, b, b % 2, f, bt)
                fire_gather(c + _NBUF, b)
            return carry

        lax.fori_loop(1, _GROUPS - 1, steady, 0)

        for b in range(_NBUF):
            c = (_GROUPS - 1) * _NBUF + b
            g = g0 + c
            fq, btq = fb(g - 2)
            wait_out(b % 2, fq, btq)
            f, bt = fb(g)
            drain(c, b, b % 2, f, bt)
        for c in range(_CPW - 2, _CPW):
            g = g0 + c
            f, bt = fb(g)
            wait_out(c % 2, f, bt)

    return k


_sc_gather = _make_sc_gather()


def kernel(x, table):
    x_blocks = jnp.transpose(x.astype(jnp.int32)).reshape(_NW, _CPW, _CHUNK)
    tbl_pad = jnp.pad(table, ((0, 0), (0, _PADDED - _EMBED_DIM)))
    out5d = _sc_gather(x_blocks, tbl_pad)
    return out5d.transpose(2, 4, 0, 1, 3).reshape(
        _BATCH, _NUM_FIELDS, _EMBED_DIM
    )


# speedup vs baseline: 1.0017x; 1.0014x over previous
import functools

import jax
import jax.numpy as jnp
from jax import lax
from jax.experimental import pallas as pl
from jax.experimental.pallas import tpu as pltpu
from jax.experimental.pallas import tpu_sc as plsc

_NUM_FIELDS = 26
_FIELD_SIZE = 100000
_EMBED_DIM = 32
_BATCH = 16384
_TOTAL = _BATCH * _NUM_FIELDS
_CHUNK = 128
_NCHUNKS = _TOTAL // _CHUNK
_NC = 2
_NS = 16
_NW = _NC * _NS
_CPW = _NCHUNKS // _NW
_NBUF = 4
_GROUPS = _CPW // _NBUF
_LANES = 16
_PADDED = 128
_TABLE_ROWS = 2600000


def _make_sc_gather():
    mesh = plsc.VectorSubcoreMesh(core_axis_name="c", subcore_axis_name="s")

    @functools.partial(
        pl.kernel,
        mesh=mesh,
        out_type=jax.ShapeDtypeStruct(
            (_NUM_FIELDS, 4, 128, 8, _CHUNK), jnp.float32
        ),
        compiler_params=pltpu.CompilerParams(
            use_tc_tiling_on_sc=False, needs_layout_passes=False
        ),
        scratch_types=(
            [
                pltpu.VMEM((_CPW, _CHUNK), jnp.int32),
            ]
            + [pltpu.VMEM((_CHUNK, _EMBED_DIM), jnp.float32)] * _NBUF
            + [pltpu.VMEM((2, 32, _CHUNK), jnp.float32)]
            + [pltpu.SemaphoreType.DMA] * (_NBUF + 2)
        ),
    )
    def k(x_hbm, tbl_hbm, out_hbm, idx_v, *bufs_and_sems):
        rows_v = bufs_and_sems[:_NBUF]
        obuf_v = bufs_and_sems[_NBUF]
        sems = bufs_and_sems[_NBUF + 1:]
        gsems = sems[:_NBUF]
        osems = sems[_NBUF:]
        wid = lax.axis_index("s") * _NC + lax.axis_index("c")
        g0 = wid * _CPW
        pltpu.sync_copy(x_hbm.at[wid], idx_v)

        rowidx = [lax.iota(jnp.int32, _LANES) + _LANES * kk for kk in range(8)]
        zeros = jnp.zeros((_LANES,), jnp.int32)

        def add_offsets(c, f):
            off = zeros + f * _FIELD_SIZE
            for kk in range(_CHUNK // _LANES):
                sl = pl.ds(kk * _LANES, _LANES)
                idx_v[c, sl] = (idx_v[c, sl] + off) * 4

        def fire_gather(c, b):
            pltpu.async_copy(tbl_hbm.at[idx_v.at[c]], rows_v[b], gsems[b])

        def wait_gather(c, b):
            pltpu.make_async_copy(
                tbl_hbm.at[idx_v.at[c]], rows_v[b], gsems[b]
            ).wait()

        def drain(c, b, p, f, bt):
            wait_gather(c, b)

            def tj(j, carry):
                col = zeros + j
                for kk in range(8):
                    v = plsc.load_gather(rows_v[b], [rowidx[kk], col])
                    obuf_v[p, j, pl.ds(kk * _LANES, _LANES)] = v
                return carry

            lax.fori_loop(0, 32, tj, 0)
            for cg in range(4):
                pltpu.async_copy(
                    obuf_v.at[p, pl.ds(8 * cg, 8)],
                    out_hbm.at[f, cg, bt],
                    osems[p],
                )

        def wait_out(p, f, bt):
            for cg in range(4):
                pltpu.make_async_copy(
                    obuf_v.at[p, pl.ds(8 * cg, 8)],
                    out_hbm.at[f, cg, bt],
                    osems[p],
                ).wait()

        def fb(g):
            return g // _CHUNK, lax.rem(g, _CHUNK)

        for b in range(_NBUF):
            add_offsets(b, (g0 + b) // _CHUNK)
            fire_gather(b, b)

        for c in range(_NBUF):
            g = g0 + c
            if c >= 2:
                fq, btq = fb(g - 2)
                wait_out(c % 2, fq, btq)
            add_offsets(c + _NBUF, (g + _NBUF) // _CHUNK)
            f, bt = fb(g)
            drain(c, c % _NBUF, c % 2, f, bt)
            fire_gather(c + _NBUF, c % _NBUF)

        def steady(gr, carry):
            for b in range(_NBUF):
                c = gr * _NBUF + b
                g = g0 + c
                fq, btq = fb(g - 2)
                wait_out(b % 2, fq, btq)
                add_offsets(c + _NBUF, (g + _NBUF) // _CHUNK)
                f, bt = fb(g)
                drain(c, b, b % 2, f, bt)
                fire_gather(c + _NBUF, b)
            return carry

        lax.fori_loop(1, _GROUPS - 1, steady, 0)

        for b in range(_NBUF):
            c = (_GROUPS - 1) * _NBUF + b
            g = g0 + c
            fq, btq = fb(g - 2)
            wait_out(b % 2, fq, btq)
            f, bt = fb(g)
            drain(c, b, b % 2, f, bt)
        for c in range(_CPW - 2, _CPW):
            g = g0 + c
            f, bt = fb(g)
            wait_out(c % 2, f, bt)

    return k


_sc_gather = _make_sc_gather()


def kernel(x, table):
    x_blocks = jnp.transpose(x.astype(jnp.int32)).reshape(_NW, _CPW, _CHUNK)
    tbl_pad = jnp.pad(table, ((0, 0), (0, _PADDED - _EMBED_DIM)))
    tbl_rows = tbl_pad.reshape(4 * _TABLE_ROWS, _EMBED_DIM)
    out5d = _sc_gather(x_blocks, tbl_rows)
    return out5d.transpose(2, 4, 0, 1, 3).reshape(
        _BATCH, _NUM_FIELDS, _EMBED_DIM
    )
